# baseline (speedup 1.0000x reference)
import functools

import jax
import jax.numpy as jnp
from jax import lax
from jax.experimental import pallas as pl
from jax.experimental.pallas import tpu as pltpu
from jax.experimental.pallas import tpu_sc as plsc

_NUM_CORES = 2
_NUM_SUBCORES = 16
_NW = _NUM_CORES * _NUM_SUBCORES
_SLAB = 16
_CHUNK = 64
_L = 16


def _sc_gather_t(table_s, idx, dims, n_nodes, n_idx):
    per_w = n_idx // _NW
    n_chunks = per_w // _CHUNK
    rows_per_dim = n_nodes // _SLAB
    dh = dims // 2
    mesh = plsc.VectorSubcoreMesh(core_axis_name="c", subcore_axis_name="s")

    @functools.partial(
        pl.kernel,
        mesh=mesh,
        out_type=jax.ShapeDtypeStruct((dims, n_idx), jnp.float32),
        scratch_types=[
            pltpu.VMEM((per_w,), jnp.int32),
            pltpu.VMEM((_CHUNK,), jnp.int32),
            pltpu.VMEM((_CHUNK,), jnp.int32),
            pltpu.VMEM((2, dh, _CHUNK), jnp.int32),
            pltpu.VMEM((2, dh, _CHUNK, _SLAB), jnp.float32),
            pltpu.VMEM((dims, _CHUNK), jnp.float32),
            pltpu.SemaphoreType.DMA,
            pltpu.SemaphoreType.DMA,
        ],
        compiler_params=pltpu.CompilerParams(
            use_tc_tiling_on_sc=False, needs_layout_passes=False
        ),
    )
    def gather_kernel(tab_hbm, idx_hbm, out_hbm, idx_v, nsh_v, off_v,
                      idxd_v, slab_v, outc_v, sem0, sem1):
        wid = lax.axis_index("s") * _NUM_CORES + lax.axis_index("c")
        base = wid * per_w
        pltpu.sync_copy(idx_hbm.at[pl.ds(base, per_w)], idx_v)
        sems = (sem0, sem1)

        def fire_half(c, h):
            sem = sems[h]
            for dl in range(dh):
                d = h * dh + dl
                for v in range(_CHUNK // _L):
                    sl = pl.ds(v * _L, _L)
                    idxd_v[h, dl, sl] = nsh_v[sl] + d * rows_per_dim
                pltpu.async_copy(
                    tab_hbm.at[idxd_v.at[h, dl]],
                    slab_v.at[h, dl],
                    sem,
                )

        def drain_half(h):
            sem = sems[h]
            for dl in range(dh):
                pltpu.make_async_copy(
                    tab_hbm.at[idxd_v.at[h, dl]], slab_v.at[h, dl], sem
                ).wait()

        def extract_half(h):
            for dl in range(dh):
                d = h * dh + dl
                dl_vec = lax.full((_L,), dl, jnp.int32)
                for v in range(_CHUNK // _L):
                    sl = pl.ds(v * _L, _L)
                    entry = lax.iota(jnp.int32, _L) + v * _L
                    vals = plsc.load_gather(
                        slab_v.at[h], [dl_vec, entry, off_v[sl]]
                    )
                    outc_v[d, sl] = vals

        def prep_chunk(c):
            for v in range(_CHUNK // _L):
                sl = pl.ds(c * _CHUNK + v * _L, _L)
                n = idx_v[sl]
                dsl = pl.ds(v * _L, _L)
                nsh_v[dsl] = lax.shift_right_logical(n, 4)
                off_v[dsl] = lax.bitwise_and(n, 15)

        def flush_chunk(c):
            pltpu.sync_copy(
                outc_v, out_hbm.at[:, pl.ds(base + c * _CHUNK, _CHUNK)]
            )

        prep_chunk(0)
        fire_half(0, 0)
        state = 0

        def chunk_body(c, state):
            fire_half(c, 1)
            drain_half(0)
            extract_half(0)
            drain_half(1)
            extract_half(1)
            flush_chunk(c)

            @pl.when(c + 1 < n_chunks)
            def _():
                prep_chunk(c + 1)
                fire_half(c + 1, 0)

            return state

        lax.fori_loop(0, n_chunks, chunk_body, state)

    return gather_kernel(table_s, idx)


def _mlp_t_body(gs_ref, gd_ref, w1_ref, b1_ref, w2_ref, b2_ref, w3_ref,
                b3_ref, out_ref):
    e = gs_ref[...] + gd_ref[...]
    h = jnp.dot(w1_ref[...], e, preferred_element_type=jnp.float32)
    h = jnp.maximum(h + b1_ref[...], 0.0)
    h = jnp.dot(w2_ref[...], h, preferred_element_type=jnp.float32)
    h = jnp.maximum(h + b2_ref[...], 0.0)
    h = jnp.dot(w3_ref[...], h, preferred_element_type=jnp.float32)
    h = jnp.maximum(h + b3_ref[...], 0.0)
    m = jnp.max(h, axis=0, keepdims=True)
    ex = jnp.exp(h - m)
    sm = ex / jnp.sum(ex, axis=0, keepdims=True)
    out_ref[...] = sm.T


def _tc_mlp_t(g_t, W1t, b1, W2t, b2, W3t, b3, batch, blk):
    n_cls = W3t.shape[0]
    dims = W1t.shape[0]
    grid = batch // blk
    full = lambda a: pl.BlockSpec(a.shape, lambda i: (0,) * a.ndim)
    return pl.pallas_call(
        _mlp_t_body,
        grid=(grid,),
        in_specs=[
            pl.BlockSpec((dims, blk), lambda i: (0, i)),
            pl.BlockSpec((dims, blk), lambda i: (0, i + grid)),
            full(W1t), full(b1), full(W2t), full(b2), full(W3t), full(b3),
        ],
        out_specs=pl.BlockSpec((blk, n_cls), lambda i: (i, 0)),
        out_shape=jax.ShapeDtypeStruct((batch, n_cls), jnp.float32),
    )(g_t, g_t, W1t, b1, W2t, b2, W3t, b3)


def kernel(src, dst, table, W1, b1, W2, b2, W3, b3):
    batch = src.shape[0]
    n_nodes, dims = table.shape
    table_s = table.T.reshape(dims * n_nodes // _SLAB, _SLAB)
    idx = jnp.concatenate([src.astype(jnp.int32), dst.astype(jnp.int32)])
    g_t = _sc_gather_t(table_s, idx, dims, n_nodes, 2 * batch)
    return _tc_mlp_t(
        g_t,
        W1.T, b1.reshape(-1, 1),
        W2.T, b2.reshape(-1, 1),
        W3.T, b3.reshape(-1, 1),
        batch, blk=2048,
    )


# speedup vs baseline: 18.7706x; 18.7706x over previous
import functools

import jax
import jax.numpy as jnp
from jax import lax
from jax.experimental import pallas as pl
from jax.experimental.pallas import tpu as pltpu
from jax.experimental.pallas import tpu_sc as plsc

_NUM_CORES = 2
_NUM_SUBCORES = 16
_NW = _NUM_CORES * _NUM_SUBCORES
_CHUNK = 128
_L = 16


def _pack_body(a_ref, b_ref, out_ref):
    out_ref[:, 0:64] = a_ref[...].T
    out_ref[:, 64:128] = b_ref[...].T


def _tc_pack_transpose(tabT, n_half, nb, dims):
    n_blocks = n_half // nb
    return pl.pallas_call(
        _pack_body,
        grid=(n_blocks,),
        in_specs=[
            pl.BlockSpec((dims, nb), lambda i: (0, i)),
            pl.BlockSpec((dims, nb), lambda i: (0, i + n_blocks)),
        ],
        out_specs=pl.BlockSpec((nb, 2 * dims), lambda i: (i, 0)),
        out_shape=jax.ShapeDtypeStruct((n_half, 2 * dims), jnp.float32),
    )(tabT, tabT)


def _sc_gather(packed, idx, n_rows, n_half):
    per_w = n_rows // _NW
    stage = 512
    n_stages = per_w // stage
    mesh = plsc.VectorSubcoreMesh(core_axis_name="c", subcore_axis_name="s")

    @functools.partial(
        pl.kernel,
        mesh=mesh,
        out_type=jax.ShapeDtypeStruct((n_rows, 128), jnp.float32),
        scratch_types=[
            pltpu.VMEM((per_w,), jnp.int32),
            pltpu.VMEM((stage, 128), jnp.float32),
            pltpu.SemaphoreType.DMA,
        ],
        compiler_params=pltpu.CompilerParams(use_tc_tiling_on_sc=True),
    )
    def gather_kernel(packed_hbm, idx_hbm, out_hbm, idx_v, buf_v, sem):
        wid = lax.axis_index("s") * _NUM_CORES + lax.axis_index("c")
        base = wid * per_w
        pltpu.sync_copy(idx_hbm.at[pl.ds(base, per_w)], idx_v)
        for v in range(per_w // _L):
            sl = pl.ds(v * _L, _L)
            n = idx_v[sl]
            idx_v[sl] = jnp.where(n >= n_half, n - n_half, n)
        for st in range(n_stages):
            copies = []
            for c in range(stage // _CHUNK):
                off = st * stage + c * _CHUNK
                copies.append(
                    pltpu.async_copy(
                        packed_hbm.at[idx_v.at[pl.ds(off, _CHUNK)]],
                        buf_v.at[pl.ds(c * _CHUNK, _CHUNK)],
                        sem,
                    )
                )
            for cp in copies:
                cp.wait()
            pltpu.sync_copy(
                buf_v, out_hbm.at[pl.ds(base + st * stage, stage)]
            )

    return gather_kernel(packed, idx)


def _mlp_body(gs_ref, gd_ref, is_ref, id_ref, w1_ref, b1_ref, w2_ref,
              b2_ref, w3_ref, b3_ref, out_ref, *, n_half):
    ss = (is_ref[...] >= n_half).astype(jnp.float32)
    sd = (id_ref[...] >= n_half).astype(jnp.float32)
    gs = gs_ref[...]
    gd = gd_ref[...]
    es = gs[:, 0:64] * (1.0 - ss) + gs[:, 64:128] * ss
    ed = gd[:, 0:64] * (1.0 - sd) + gd[:, 64:128] * sd
    e = es + ed
    h = jnp.dot(e, w1_ref[...], preferred_element_type=jnp.float32)
    h = jnp.maximum(h + b1_ref[...], 0.0)
    h = jnp.dot(h, w2_ref[...], preferred_element_type=jnp.float32)
    h = jnp.maximum(h + b2_ref[...], 0.0)
    h = jnp.dot(h, w3_ref[...], preferred_element_type=jnp.float32)
    h = jnp.maximum(h + b3_ref[...], 0.0)
    m = jnp.max(h, axis=1, keepdims=True)
    ex = jnp.exp(h - m)
    out_ref[...] = ex / jnp.sum(ex, axis=1, keepdims=True)


def _tc_mlp(g, idx2, W1, b1, W2, b2, W3, b3, batch, blk, n_half):
    n_cls = W3.shape[1]
    grid = batch // blk
    full = lambda a: pl.BlockSpec(a.shape, lambda i: (0,) * a.ndim)
    return pl.pallas_call(
        functools.partial(_mlp_body, n_half=n_half),
        grid=(grid,),
        in_specs=[
            pl.BlockSpec((blk, 128), lambda i: (i, 0)),
            pl.BlockSpec((blk, 128), lambda i: (i + grid, 0)),
            pl.BlockSpec((blk, 1), lambda i: (i, 0)),
            pl.BlockSpec((blk, 1), lambda i: (i + grid, 0)),
            full(W1), full(b1), full(W2), full(b2), full(W3), full(b3),
        ],
        out_specs=pl.BlockSpec((blk, n_cls), lambda i: (i, 0)),
        out_shape=jax.ShapeDtypeStruct((batch, n_cls), jnp.float32),
    )(g, g, idx2, idx2, W1, b1, W2, b2, W3, b3)


def kernel(src, dst, table, W1, b1, W2, b2, W3, b3):
    batch = src.shape[0]
    n_nodes, dims = table.shape
    nb = 16384
    n_half = ((n_nodes // 2 + nb - 1) // nb) * nb
    packed = _tc_pack_transpose(table.T, n_half, nb=nb, dims=dims)
    idx = jnp.concatenate([src.astype(jnp.int32), dst.astype(jnp.int32)])
    g = _sc_gather(packed, idx, 2 * batch, n_half)
    return _tc_mlp(
        g, idx.reshape(-1, 1),
        W1, b1.reshape(1, -1),
        W2, b2.reshape(1, -1),
        W3, b3.reshape(1, -1),
        batch, blk=2048, n_half=n_half,
    )


# speedup vs baseline: 19.0419x; 1.0145x over previous
import functools

import jax
import jax.numpy as jnp
from jax import lax
from jax.experimental import pallas as pl
from jax.experimental.pallas import tpu as pltpu
from jax.experimental.pallas import tpu_sc as plsc

_NUM_CORES = 2
_NUM_SUBCORES = 16
_NW = _NUM_CORES * _NUM_SUBCORES
_CHUNK = 128
_L = 16

_TDIMS = (((0,), (0,)), ((), ()))


def _pack_body(a_ref, b_ref, w1_ref, out_ref):
    w1 = w1_ref[...]
    pa = lax.dot_general(a_ref[...], w1, _TDIMS,
                         preferred_element_type=jnp.float32)
    pb = lax.dot_general(b_ref[...], w1, _TDIMS,
                         preferred_element_type=jnp.float32)
    out_ref[...] = jnp.concatenate([pa, pb], axis=1)


def _tc_pack_w1(tabT, W1, n_half, nb, dims):
    n_blocks = n_half // nb
    return pl.pallas_call(
        _pack_body,
        grid=(n_blocks,),
        in_specs=[
            pl.BlockSpec((dims, nb), lambda i: (0, i)),
            pl.BlockSpec((dims, nb), lambda i: (0, i + n_blocks)),
            pl.BlockSpec(W1.shape, lambda i: (0, 0)),
        ],
        out_specs=pl.BlockSpec((nb, 2 * dims), lambda i: (i, 0)),
        out_shape=jax.ShapeDtypeStruct((n_half, 2 * dims), jnp.float32),
    )(tabT, tabT, W1)


def _sc_gather(packed, idx, n_rows, n_half):
    per_w = n_rows // _NW
    stage = 512
    n_stages = per_w // stage
    mesh = plsc.VectorSubcoreMesh(core_axis_name="c", subcore_axis_name="s")

    @functools.partial(
        pl.kernel,
        mesh=mesh,
        out_type=jax.ShapeDtypeStruct((n_rows, 128), jnp.float32),
        scratch_types=[
            pltpu.VMEM((per_w,), jnp.int32),
            pltpu.VMEM((stage, 128), jnp.float32),
            pltpu.SemaphoreType.DMA,
        ],
        compiler_params=pltpu.CompilerParams(use_tc_tiling_on_sc=True),
    )
    def gather_kernel(packed_hbm, idx_hbm, out_hbm, idx_v, buf_v, sem):
        wid = lax.axis_index("s") * _NUM_CORES + lax.axis_index("c")
        base = wid * per_w
        pltpu.sync_copy(idx_hbm.at[pl.ds(base, per_w)], idx_v)
        for v in range(per_w // _L):
            sl = pl.ds(v * _L, _L)
            n = idx_v[sl]
            idx_v[sl] = jnp.where(n >= n_half, n - n_half, n)
        for st in range(n_stages):
            copies = []
            for c in range(stage // _CHUNK):
                off = st * stage + c * _CHUNK
                copies.append(
                    pltpu.async_copy(
                        packed_hbm.at[idx_v.at[pl.ds(off, _CHUNK)]],
                        buf_v.at[pl.ds(c * _CHUNK, _CHUNK)],
                        sem,
                    )
                )
            for cp in copies:
                cp.wait()
            pltpu.sync_copy(
                buf_v, out_hbm.at[pl.ds(base + st * stage, stage)]
            )

    return gather_kernel(packed, idx)


def _mlp_body(gs_ref, gd_ref, is_ref, id_ref, b1_ref, w2_ref,
              b2_ref, w3_ref, b3_ref, out_ref, *, n_half):
    ss = (is_ref[...] >= n_half).astype(jnp.float32)
    sd = (id_ref[...] >= n_half).astype(jnp.float32)
    gs = gs_ref[...]
    gd = gd_ref[...]
    es = gs[:, 0:64] * (1.0 - ss) + gs[:, 64:128] * ss
    ed = gd[:, 0:64] * (1.0 - sd) + gd[:, 64:128] * sd
    h = jnp.maximum(es + ed + b1_ref[...], 0.0)
    h = jnp.dot(h, w2_ref[...], preferred_element_type=jnp.float32)
    h = jnp.maximum(h + b2_ref[...], 0.0)
    h = jnp.dot(h, w3_ref[...], preferred_element_type=jnp.float32)
    h = jnp.maximum(h + b3_ref[...], 0.0)
    m = jnp.max(h, axis=1, keepdims=True)
    ex = jnp.exp(h - m)
    sm = ex / jnp.sum(ex, axis=1, keepdims=True)
    out_ref[...] = sm.T


def _tc_mlp(g, idx2, b1, W2, b2, W3, b3, batch, blk, n_half):
    n_cls = W3.shape[1]
    grid = batch // blk
    full = lambda a: pl.BlockSpec(a.shape, lambda i: (0,) * a.ndim)
    return pl.pallas_call(
        functools.partial(_mlp_body, n_half=n_half),
        grid=(grid,),
        in_specs=[
            pl.BlockSpec((blk, 128), lambda i: (i, 0)),
            pl.BlockSpec((blk, 128), lambda i: (i + grid, 0)),
            pl.BlockSpec((blk, 1), lambda i: (i, 0)),
            pl.BlockSpec((blk, 1), lambda i: (i + grid, 0)),
            full(b1), full(W2), full(b2), full(W3), full(b3),
        ],
        out_specs=pl.BlockSpec((n_cls, blk), lambda i: (0, i)),
        out_shape=jax.ShapeDtypeStruct((n_cls, batch), jnp.float32),
    )(g, g, idx2, idx2, b1, W2, b2, W3, b3)


def kernel(src, dst, table, W1, b1, W2, b2, W3, b3):
    batch = src.shape[0]
    n_nodes, dims = table.shape
    nb = 16384
    n_half = ((n_nodes // 2 + nb - 1) // nb) * nb
    packed = _tc_pack_w1(table.T, W1, n_half, nb=nb, dims=dims)
    idx = jnp.concatenate([src.astype(jnp.int32), dst.astype(jnp.int32)])
    g = _sc_gather(packed, idx, 2 * batch, n_half)
    out_t = _tc_mlp(
        g, idx.reshape(-1, 1),
        b1.reshape(1, -1),
        W2, b2.reshape(1, -1),
        W3, b3.reshape(1, -1),
        batch, blk=2048, n_half=n_half,
    )
    return out_t.T
